# baseline (speedup 1.0000x reference)
import jax
import jax.numpy as jnp
from jax.experimental import pallas as pl
from jax.experimental.pallas import tpu as pltpu

_ALPHA = 0.25
_GAMMA = 2.0

_B, _N, _M, _C = 4, 50000, 32, 80
_T = 2000
_NT = _N // _T


def _loss_body(cls_ref, bbox_ref, anc_ref, boxes_ref, labels_ref, out_ref, acc):
    b = pl.program_id(0)
    j = pl.program_id(1)

    @pl.when(j == 0)
    def _():
        acc[0] = 0.0
        acc[1] = 0.0
        acc[2] = 0.0

    a = anc_ref[...]
    acx = a[:, 0:1]
    acy = a[:, 1:2]
    aw = a[:, 2:3]
    ah = a[:, 3:4]
    al = acx - aw * 0.5
    at_ = acy - ah * 0.5
    ar = acx + aw * 0.5
    ab = acy + ah * 0.5

    bx = boxes_ref[...]
    bcx = bx[0:1, :]
    bcy = bx[1:2, :]
    bw = bx[2:3, :]
    bh = bx[3:4, :]
    bl = bcx - bw * 0.5
    bt = bcy - bh * 0.5
    br = bcx + bw * 0.5
    bb = bcy + bh * 0.5

    ix0 = jnp.maximum(al, bl)
    iy0 = jnp.maximum(at_, bt)
    ix1 = jnp.minimum(ar, br)
    iy1 = jnp.minimum(ab, bb)
    iw = jnp.clip(ix1 - ix0, 0.0, None)
    ih = jnp.clip(iy1 - iy0, 0.0, None)
    inter = iw * ih
    area_a = (ar - al) * (ab - at_)
    area_b = (br - bl) * (bb - bt)
    union = jnp.maximum(area_a + area_b - inter, 1e-12)
    iou = inter / union

    vals = jnp.max(iou, axis=1, keepdims=True)
    ids = jax.lax.broadcasted_iota(jnp.int32, (_T, _M), 1)
    idx = jnp.min(jnp.where(iou == vals, ids, _M), axis=1, keepdims=True)

    fg = vals > 0.5
    fg_f = fg.astype(jnp.float32)
    clas_mask_f = (fg | (vals < 0.4)).astype(jnp.float32)

    safe = jnp.where(fg, idx, 0)
    onehot = (safe == ids).astype(jnp.float32)

    tgt_cx = jnp.sum(onehot * bcx, axis=1, keepdims=True)
    tgt_cy = jnp.sum(onehot * bcy, axis=1, keepdims=True)
    tgt_w = jnp.sum(onehot * bw, axis=1, keepdims=True)
    tgt_h = jnp.sum(onehot * bh, axis=1, keepdims=True)
    lab_row = labels_ref[...]
    lab_sel = jnp.sum(onehot * lab_row, axis=1, keepdims=True) * fg_f

    t_cx = (tgt_cx - acx) / aw * 10.0
    t_cy = (tgt_cy - acy) / ah * 10.0
    t_w = jnp.log(tgt_w / aw + 1e-8) * 5.0
    t_h = jnp.log(tgt_h / ah + 1e-8) * 5.0

    p = bbox_ref[...]
    d0 = jnp.abs(p[:, 0:1] - t_cx)
    d1 = jnp.abs(p[:, 1:2] - t_cy)
    d2 = jnp.abs(p[:, 2:3] - t_w)
    d3 = jnp.abs(p[:, 3:4] - t_h)

    def sl1(d):
        return jnp.where(d < 1.0, 0.5 * d * d, d - 0.5)

    sl1_tile = jnp.sum((sl1(d0) + sl1(d1) + sl1(d2) + sl1(d3)) * fg_f)

    x = cls_ref[...]
    s = jax.nn.sigmoid(x)
    sp = jnp.maximum(x, 0.0) + jnp.log1p(jnp.exp(-jnp.abs(x)))
    cls_ids = jax.lax.broadcasted_iota(jnp.int32, (_T, _C), 1).astype(jnp.float32)
    t = ((cls_ids == lab_sel - 1.0) & (lab_sel > 0.5)).astype(jnp.float32)
    w = t * (1.0 - s) + (1.0 - t) * s
    alpha = (1.0 - t) * _ALPHA + t * (1.0 - _ALPHA)
    weights = w * w * alpha
    bce = sp - x * t
    focal_tile = jnp.sum(bce * weights * clas_mask_f)

    acc[0] += jnp.sum(fg_f)
    acc[1] += sl1_tile
    acc[2] += focal_tile

    @pl.when(jnp.logical_and(b == 0, j == 0))
    def _():
        out_ref[0] = 0.0
        out_ref[1] = 0.0

    @pl.when(j == _NT - 1)
    def _():
        n_fg = acc[0]
        bb_loss = jnp.where(n_fg > 0.0,
                            acc[1] / jnp.maximum(n_fg * 4.0, 1.0), 0.0)
        cls_loss = acc[2] / jnp.maximum(n_fg, 1.0)
        out_ref[0] += cls_loss / _B
        out_ref[1] += bb_loss / _B


def kernel(cls_preds, bbox_preds, anchors, boxes, labels):
    boxes_t = jnp.transpose(boxes, (0, 2, 1))
    labels_f = labels.astype(jnp.float32).reshape(_B, 1, _M)

    out = pl.pallas_call(
        _loss_body,
        grid=(_B, _NT),
        in_specs=[
            pl.BlockSpec((None, _T, _C), lambda b, j: (b, j, 0)),
            pl.BlockSpec((None, _T, 4), lambda b, j: (b, j, 0)),
            pl.BlockSpec((None, _T, 4), lambda b, j: (b, j, 0)),
            pl.BlockSpec((None, 4, _M), lambda b, j: (b, 0, 0)),
            pl.BlockSpec((None, 1, _M), lambda b, j: (b, 0, 0)),
        ],
        out_specs=pl.BlockSpec(memory_space=pltpu.SMEM),
        out_shape=jax.ShapeDtypeStruct((2,), jnp.float32),
        scratch_shapes=[pltpu.SMEM((3,), jnp.float32)],
    )(cls_preds, bbox_preds, anchors, boxes_t, labels_f)
    return out


# speedup vs baseline: 3.6732x; 3.6732x over previous
import jax
import jax.numpy as jnp
from jax.experimental import pallas as pl
from jax.experimental.pallas import tpu as pltpu

_ALPHA = 0.25
_B, _N, _M, _C = 4, 50000, 32, 80
_T = 2000
_NT = _N // _T

_DN = (((1,), (0,)), ((), ()))


def _loss_body(cls_ref, bbox_ref, anc_ref, boxes_ref, boxes_t_ref,
               labels_ref, out_ref, acc):
    b = pl.program_id(0)
    j = pl.program_id(1)

    @pl.when(j == 0)
    def _():
        acc[0] = 0.0
        acc[1] = 0.0
        acc[2] = 0.0

    a = anc_ref[...]
    acx = a[0:1, :]
    acy = a[1:2, :]
    aw = a[2:3, :]
    ah = a[3:4, :]
    al = acx - aw * 0.5
    at_ = acy - ah * 0.5
    ar = acx + aw * 0.5
    ab = acy + ah * 0.5

    bx = boxes_ref[...]
    bcx = bx[:, 0:1]
    bcy = bx[:, 1:2]
    bw = bx[:, 2:3]
    bh = bx[:, 3:4]
    bl = bcx - bw * 0.5
    bt = bcy - bh * 0.5
    br = bcx + bw * 0.5
    bb = bcy + bh * 0.5

    ix0 = jnp.maximum(al, bl)
    iy0 = jnp.maximum(at_, bt)
    ix1 = jnp.minimum(ar, br)
    iy1 = jnp.minimum(ab, bb)
    iw = jnp.clip(ix1 - ix0, 0.0, None)
    ih = jnp.clip(iy1 - iy0, 0.0, None)
    inter = iw * ih
    area_a = (ar - al) * (ab - at_)
    area_b = (br - bl) * (bb - bt)
    union = jnp.maximum(area_a + area_b - inter, 1e-12)
    iou = inter / union

    vals = jnp.max(iou, axis=0, keepdims=True)
    ids32 = jax.lax.broadcasted_iota(jnp.int32, (_M, _T), 0)
    idx = jnp.min(jnp.where(iou == vals, ids32, _M), axis=0, keepdims=True)

    fg = vals > 0.5
    fg_f = fg.astype(jnp.float32)
    cm_row = (fg | (vals < 0.4)).astype(jnp.float32)

    safe2 = jnp.where(fg, idx, -1)
    oh32 = (ids32 == safe2).astype(jnp.float32)

    tgt4 = jax.lax.dot_general(boxes_t_ref[...], oh32, _DN,
                               preferred_element_type=jnp.float32)

    p = bbox_ref[...]
    t_cx = (tgt4[0:1, :] - acx) / aw * 10.0
    t_cy = (tgt4[1:2, :] - acy) / ah * 10.0
    t_w = jnp.log(tgt4[2:3, :] / aw + 1e-8) * 5.0
    t_h = jnp.log(tgt4[3:4, :] / ah + 1e-8) * 5.0

    def sl1(d):
        return jnp.where(d < 1.0, 0.5 * d * d, d - 0.5)

    sl1_row = (sl1(jnp.abs(p[0:1, :] - t_cx)) + sl1(jnp.abs(p[1:2, :] - t_cy))
               + sl1(jnp.abs(p[2:3, :] - t_w)) + sl1(jnp.abs(p[3:4, :] - t_h)))
    sl1_tile = jnp.sum(sl1_row * fg_f)

    x = cls_ref[...]
    u = jnp.exp(-jnp.abs(x))
    opu = 1.0 + u
    lg = jnp.log(opu)
    sp = jnp.maximum(x, 0.0) + lg
    r = 1.0 / opu
    s = jnp.where(x >= 0.0, r, u * r)
    l0 = sp * (s * s)
    l1 = (sp - x) * ((1.0 - s) * (1.0 - s))

    term1 = jnp.sum(jax.lax.dot_general(cm_row, l0, _DN,
                                        preferred_element_type=jnp.float32))
    p0 = jax.lax.dot_general(oh32, l0, _DN,
                             preferred_element_type=jnp.float32)
    p1 = jax.lax.dot_general(oh32, l1, _DN,
                             preferred_element_type=jnp.float32)
    labs = labels_ref[...]
    cls_ids = jax.lax.broadcasted_iota(jnp.int32, (_M, _C), 1)
    labmat = (labs == cls_ids.astype(jnp.float32) + 1.0).astype(jnp.float32)
    corr1 = jnp.sum(p1 * labmat)
    corr0 = jnp.sum(p0 * labmat)
    focal_tile = 0.25 * (term1 - corr0) + 0.75 * corr1

    acc[0] += jnp.sum(fg_f)
    acc[1] += sl1_tile
    acc[2] += focal_tile

    @pl.when(jnp.logical_and(b == 0, j == 0))
    def _():
        out_ref[0] = 0.0
        out_ref[1] = 0.0

    @pl.when(j == _NT - 1)
    def _():
        n_fg = acc[0]
        bb_loss = jnp.where(n_fg > 0.0,
                            acc[1] / jnp.maximum(n_fg * 4.0, 1.0), 0.0)
        cls_loss = acc[2] / jnp.maximum(n_fg, 1.0)
        out_ref[0] += cls_loss / _B
        out_ref[1] += bb_loss / _B


def kernel(cls_preds, bbox_preds, anchors, boxes, labels):
    bbox_t = jnp.transpose(bbox_preds.reshape(_B, _NT, _T, 4), (0, 1, 3, 2))
    anchors_t = jnp.transpose(anchors.reshape(_B, _NT, _T, 4), (0, 1, 3, 2))
    boxes_t = jnp.transpose(boxes, (0, 2, 1))
    labels_f = labels.astype(jnp.float32).reshape(_B, _M, 1)

    out = pl.pallas_call(
        _loss_body,
        grid=(_B, _NT),
        in_specs=[
            pl.BlockSpec((None, _T, _C), lambda b, j: (b, j, 0)),
            pl.BlockSpec((None, None, 4, _T), lambda b, j: (b, j, 0, 0)),
            pl.BlockSpec((None, None, 4, _T), lambda b, j: (b, j, 0, 0)),
            pl.BlockSpec((None, _M, 4), lambda b, j: (b, 0, 0)),
            pl.BlockSpec((None, 4, _M), lambda b, j: (b, 0, 0)),
            pl.BlockSpec((None, _M, 1), lambda b, j: (b, 0, 0)),
        ],
        out_specs=pl.BlockSpec(memory_space=pltpu.SMEM),
        out_shape=jax.ShapeDtypeStruct((2,), jnp.float32),
        scratch_shapes=[pltpu.SMEM((3,), jnp.float32)],
    )(cls_preds, bbox_t, anchors_t, boxes, boxes_t, labels_f)
    return out


# speedup vs baseline: 3.9909x; 1.0865x over previous
import jax
import jax.numpy as jnp
from jax.experimental import pallas as pl
from jax.experimental.pallas import tpu as pltpu

_ALPHA = 0.25
_B, _N, _M, _C = 4, 50000, 32, 80
_T = 5000
_NT = _N // _T

_DN = (((1,), (0,)), ((), ()))


def _loss_body(cls_ref, bbox_ref, anc_ref, boxes_ref, boxes_t_ref,
               labels_ref, out_ref, acc):
    b = pl.program_id(0)
    j = pl.program_id(1)

    @pl.when(j == 0)
    def _():
        acc[0] = 0.0
        acc[1] = 0.0
        acc[2] = 0.0

    a = anc_ref[...]
    acx = a[0:1, :]
    acy = a[1:2, :]
    aw = a[2:3, :]
    ah = a[3:4, :]
    al = acx - aw * 0.5
    at_ = acy - ah * 0.5
    ar = acx + aw * 0.5
    ab = acy + ah * 0.5

    bx = boxes_ref[...]
    bcx = bx[:, 0:1]
    bcy = bx[:, 1:2]
    bw = bx[:, 2:3]
    bh = bx[:, 3:4]
    bl = bcx - bw * 0.5
    bt = bcy - bh * 0.5
    br = bcx + bw * 0.5
    bb = bcy + bh * 0.5

    ix0 = jnp.maximum(al, bl)
    iy0 = jnp.maximum(at_, bt)
    ix1 = jnp.minimum(ar, br)
    iy1 = jnp.minimum(ab, bb)
    iw = jnp.clip(ix1 - ix0, 0.0, None)
    ih = jnp.clip(iy1 - iy0, 0.0, None)
    inter = iw * ih
    area_a = (ar - al) * (ab - at_)
    area_b = (br - bl) * (bb - bt)
    union = jnp.maximum(area_a + area_b - inter, 1e-12)
    iou = inter / union

    vals = jnp.max(iou, axis=0, keepdims=True)
    ids32 = jax.lax.broadcasted_iota(jnp.int32, (_M, _T), 0)
    idx = jnp.min(jnp.where(iou == vals, ids32, _M), axis=0, keepdims=True)

    fg = vals > 0.5
    fg_f = fg.astype(jnp.float32)
    cm_row = (fg | (vals < 0.4)).astype(jnp.float32)

    safe2 = jnp.where(fg, idx, -1)
    oh32 = (ids32 == safe2).astype(jnp.float32)

    tgt4 = jax.lax.dot_general(boxes_t_ref[...], oh32, _DN,
                               preferred_element_type=jnp.float32)

    p = bbox_ref[...]
    t_cx = (tgt4[0:1, :] - acx) / aw * 10.0
    t_cy = (tgt4[1:2, :] - acy) / ah * 10.0
    t_w = jnp.log(tgt4[2:3, :] / aw + 1e-8) * 5.0
    t_h = jnp.log(tgt4[3:4, :] / ah + 1e-8) * 5.0

    def sl1(d):
        return jnp.where(d < 1.0, 0.5 * d * d, d - 0.5)

    sl1_row = (sl1(jnp.abs(p[0:1, :] - t_cx)) + sl1(jnp.abs(p[1:2, :] - t_cy))
               + sl1(jnp.abs(p[2:3, :] - t_w)) + sl1(jnp.abs(p[3:4, :] - t_h)))
    sl1_tile = jnp.sum(sl1_row * fg_f)

    x = cls_ref[...]
    u = jnp.exp(-jnp.abs(x))
    opu = 1.0 + u
    lg = jnp.log(opu)
    sp = jnp.maximum(x, 0.0) + lg
    r = 1.0 / opu
    s = jnp.where(x >= 0.0, r, u * r)
    l0 = sp * (s * s)
    l1 = (sp - x) * ((1.0 - s) * (1.0 - s))

    term1 = jnp.sum(jax.lax.dot_general(cm_row, l0, _DN,
                                        preferred_element_type=jnp.float32))
    p0 = jax.lax.dot_general(oh32, l0, _DN,
                             preferred_element_type=jnp.float32)
    p1 = jax.lax.dot_general(oh32, l1, _DN,
                             preferred_element_type=jnp.float32)
    labs = labels_ref[...]
    cls_ids = jax.lax.broadcasted_iota(jnp.int32, (_M, _C), 1)
    labmat = (labs == cls_ids.astype(jnp.float32) + 1.0).astype(jnp.float32)
    corr1 = jnp.sum(p1 * labmat)
    corr0 = jnp.sum(p0 * labmat)
    focal_tile = 0.25 * (term1 - corr0) + 0.75 * corr1

    acc[0] += jnp.sum(fg_f)
    acc[1] += sl1_tile
    acc[2] += focal_tile

    @pl.when(jnp.logical_and(b == 0, j == 0))
    def _():
        out_ref[0] = 0.0
        out_ref[1] = 0.0

    @pl.when(j == _NT - 1)
    def _():
        n_fg = acc[0]
        bb_loss = jnp.where(n_fg > 0.0,
                            acc[1] / jnp.maximum(n_fg * 4.0, 1.0), 0.0)
        cls_loss = acc[2] / jnp.maximum(n_fg, 1.0)
        out_ref[0] += cls_loss / _B
        out_ref[1] += bb_loss / _B


def kernel(cls_preds, bbox_preds, anchors, boxes, labels):
    bbox_t = jnp.transpose(bbox_preds.reshape(_B, _NT, _T, 4), (0, 1, 3, 2))
    anchors_t = jnp.transpose(anchors.reshape(_B, _NT, _T, 4), (0, 1, 3, 2))
    boxes_t = jnp.transpose(boxes, (0, 2, 1))
    labels_f = labels.astype(jnp.float32).reshape(_B, _M, 1)

    out = pl.pallas_call(
        _loss_body,
        grid=(_B, _NT),
        in_specs=[
            pl.BlockSpec((None, _T, _C), lambda b, j: (b, j, 0)),
            pl.BlockSpec((None, None, 4, _T), lambda b, j: (b, j, 0, 0)),
            pl.BlockSpec((None, None, 4, _T), lambda b, j: (b, j, 0, 0)),
            pl.BlockSpec((None, _M, 4), lambda b, j: (b, 0, 0)),
            pl.BlockSpec((None, 4, _M), lambda b, j: (b, 0, 0)),
            pl.BlockSpec((None, _M, 1), lambda b, j: (b, 0, 0)),
        ],
        out_specs=pl.BlockSpec(memory_space=pltpu.SMEM),
        out_shape=jax.ShapeDtypeStruct((2,), jnp.float32),
        scratch_shapes=[pltpu.SMEM((3,), jnp.float32)],
    )(cls_preds, bbox_t, anchors_t, boxes, boxes_t, labels_f)
    return out


# speedup vs baseline: 4.2255x; 1.0588x over previous
import jax
import jax.numpy as jnp
from jax.experimental import pallas as pl
from jax.experimental.pallas import tpu as pltpu

_ALPHA = 0.25
_B, _N, _M, _C = 4, 50000, 32, 80
_T = 10000
_NT = _N // _T

_DN = (((1,), (0,)), ((), ()))


def _loss_body(cls_ref, bbox_ref, anc_ref, boxes_ref, boxes_t_ref,
               labels_ref, out_ref, acc):
    b = pl.program_id(0)
    j = pl.program_id(1)

    @pl.when(j == 0)
    def _():
        acc[0] = 0.0
        acc[1] = 0.0
        acc[2] = 0.0

    a = anc_ref[...]
    acx = a[0:1, :]
    acy = a[1:2, :]
    aw = a[2:3, :]
    ah = a[3:4, :]
    al = acx - aw * 0.5
    at_ = acy - ah * 0.5
    ar = acx + aw * 0.5
    ab = acy + ah * 0.5

    bx = boxes_ref[...]
    bcx = bx[:, 0:1]
    bcy = bx[:, 1:2]
    bw = bx[:, 2:3]
    bh = bx[:, 3:4]
    bl = bcx - bw * 0.5
    bt = bcy - bh * 0.5
    br = bcx + bw * 0.5
    bb = bcy + bh * 0.5

    ix0 = jnp.maximum(al, bl)
    iy0 = jnp.maximum(at_, bt)
    ix1 = jnp.minimum(ar, br)
    iy1 = jnp.minimum(ab, bb)
    iw = jnp.clip(ix1 - ix0, 0.0, None)
    ih = jnp.clip(iy1 - iy0, 0.0, None)
    inter = iw * ih
    area_a = (ar - al) * (ab - at_)
    area_b = (br - bl) * (bb - bt)
    union = jnp.maximum(area_a + area_b - inter, 1e-12)
    iou = inter / union

    vals = jnp.max(iou, axis=0, keepdims=True)
    ids32 = jax.lax.broadcasted_iota(jnp.int32, (_M, _T), 0)
    idx = jnp.min(jnp.where(iou == vals, ids32, _M), axis=0, keepdims=True)

    fg = vals > 0.5
    fg_f = fg.astype(jnp.float32)
    cm_row = (fg | (vals < 0.4)).astype(jnp.float32)

    safe2 = jnp.where(fg, idx, -1)
    oh32 = (ids32 == safe2).astype(jnp.float32)

    tgt4 = jax.lax.dot_general(boxes_t_ref[...], oh32, _DN,
                               preferred_element_type=jnp.float32)

    p = bbox_ref[...]
    t_cx = (tgt4[0:1, :] - acx) / aw * 10.0
    t_cy = (tgt4[1:2, :] - acy) / ah * 10.0
    t_w = jnp.log(tgt4[2:3, :] / aw + 1e-8) * 5.0
    t_h = jnp.log(tgt4[3:4, :] / ah + 1e-8) * 5.0

    def sl1(d):
        return jnp.where(d < 1.0, 0.5 * d * d, d - 0.5)

    sl1_row = (sl1(jnp.abs(p[0:1, :] - t_cx)) + sl1(jnp.abs(p[1:2, :] - t_cy))
               + sl1(jnp.abs(p[2:3, :] - t_w)) + sl1(jnp.abs(p[3:4, :] - t_h)))
    sl1_tile = jnp.sum(sl1_row * fg_f)

    x = cls_ref[...]
    u = jnp.exp(-jnp.abs(x))
    opu = 1.0 + u
    lg = jnp.log(opu)
    sp = jnp.maximum(x, 0.0) + lg
    r = 1.0 / opu
    s = jnp.where(x >= 0.0, r, u * r)
    l0 = sp * (s * s)
    l1 = (sp - x) * ((1.0 - s) * (1.0 - s))

    term1 = jnp.sum(jax.lax.dot_general(cm_row, l0, _DN,
                                        preferred_element_type=jnp.float32))
    p0 = jax.lax.dot_general(oh32, l0, _DN,
                             preferred_element_type=jnp.float32)
    p1 = jax.lax.dot_general(oh32, l1, _DN,
                             preferred_element_type=jnp.float32)
    labs = labels_ref[...]
    cls_ids = jax.lax.broadcasted_iota(jnp.int32, (_M, _C), 1)
    labmat = (labs == cls_ids.astype(jnp.float32) + 1.0).astype(jnp.float32)
    corr1 = jnp.sum(p1 * labmat)
    corr0 = jnp.sum(p0 * labmat)
    focal_tile = 0.25 * (term1 - corr0) + 0.75 * corr1

    acc[0] += jnp.sum(fg_f)
    acc[1] += sl1_tile
    acc[2] += focal_tile

    @pl.when(jnp.logical_and(b == 0, j == 0))
    def _():
        out_ref[0] = 0.0
        out_ref[1] = 0.0

    @pl.when(j == _NT - 1)
    def _():
        n_fg = acc[0]
        bb_loss = jnp.where(n_fg > 0.0,
                            acc[1] / jnp.maximum(n_fg * 4.0, 1.0), 0.0)
        cls_loss = acc[2] / jnp.maximum(n_fg, 1.0)
        out_ref[0] += cls_loss / _B
        out_ref[1] += bb_loss / _B


def kernel(cls_preds, bbox_preds, anchors, boxes, labels):
    bbox_t = jnp.transpose(bbox_preds.reshape(_B, _NT, _T, 4), (0, 1, 3, 2))
    anchors_t = jnp.transpose(anchors.reshape(_B, _NT, _T, 4), (0, 1, 3, 2))
    boxes_t = jnp.transpose(boxes, (0, 2, 1))
    labels_f = labels.astype(jnp.float32).reshape(_B, _M, 1)

    out = pl.pallas_call(
        _loss_body,
        grid=(_B, _NT),
        in_specs=[
            pl.BlockSpec((None, _T, _C), lambda b, j: (b, j, 0)),
            pl.BlockSpec((None, None, 4, _T), lambda b, j: (b, j, 0, 0)),
            pl.BlockSpec((None, None, 4, _T), lambda b, j: (b, j, 0, 0)),
            pl.BlockSpec((None, _M, 4), lambda b, j: (b, 0, 0)),
            pl.BlockSpec((None, 4, _M), lambda b, j: (b, 0, 0)),
            pl.BlockSpec((None, _M, 1), lambda b, j: (b, 0, 0)),
        ],
        out_specs=pl.BlockSpec(memory_space=pltpu.SMEM),
        out_shape=jax.ShapeDtypeStruct((2,), jnp.float32),
        scratch_shapes=[pltpu.SMEM((3,), jnp.float32)],
    )(cls_preds, bbox_t, anchors_t, boxes, boxes_t, labels_f)
    return out


# speedup vs baseline: 4.2580x; 1.0077x over previous
import jax
import jax.numpy as jnp
from jax.experimental import pallas as pl
from jax.experimental.pallas import tpu as pltpu

_ALPHA = 0.25
_B, _N, _M, _C = 4, 50000, 32, 80
_T = 25000
_NT = _N // _T

_DN = (((1,), (0,)), ((), ()))


def _loss_body(cls_ref, bbox_ref, anc_ref, boxes_ref, boxes_t_ref,
               labels_ref, out_ref, acc):
    b = pl.program_id(0)
    j = pl.program_id(1)

    @pl.when(j == 0)
    def _():
        acc[0] = 0.0
        acc[1] = 0.0
        acc[2] = 0.0

    a = anc_ref[...]
    acx = a[0:1, :]
    acy = a[1:2, :]
    aw = a[2:3, :]
    ah = a[3:4, :]
    al = acx - aw * 0.5
    at_ = acy - ah * 0.5
    ar = acx + aw * 0.5
    ab = acy + ah * 0.5

    bx = boxes_ref[...]
    bcx = bx[:, 0:1]
    bcy = bx[:, 1:2]
    bw = bx[:, 2:3]
    bh = bx[:, 3:4]
    bl = bcx - bw * 0.5
    bt = bcy - bh * 0.5
    br = bcx + bw * 0.5
    bb = bcy + bh * 0.5

    ix0 = jnp.maximum(al, bl)
    iy0 = jnp.maximum(at_, bt)
    ix1 = jnp.minimum(ar, br)
    iy1 = jnp.minimum(ab, bb)
    iw = jnp.clip(ix1 - ix0, 0.0, None)
    ih = jnp.clip(iy1 - iy0, 0.0, None)
    inter = iw * ih
    area_a = (ar - al) * (ab - at_)
    area_b = (br - bl) * (bb - bt)
    union = jnp.maximum(area_a + area_b - inter, 1e-12)
    iou = inter / union

    vals = jnp.max(iou, axis=0, keepdims=True)
    ids32 = jax.lax.broadcasted_iota(jnp.int32, (_M, _T), 0)
    idx = jnp.min(jnp.where(iou == vals, ids32, _M), axis=0, keepdims=True)

    fg = vals > 0.5
    fg_f = fg.astype(jnp.float32)
    cm_row = (fg | (vals < 0.4)).astype(jnp.float32)

    safe2 = jnp.where(fg, idx, -1)
    oh32 = (ids32 == safe2).astype(jnp.float32)

    tgt4 = jax.lax.dot_general(boxes_t_ref[...], oh32, _DN,
                               preferred_element_type=jnp.float32)

    p = bbox_ref[...]
    t_cx = (tgt4[0:1, :] - acx) / aw * 10.0
    t_cy = (tgt4[1:2, :] - acy) / ah * 10.0
    t_w = jnp.log(tgt4[2:3, :] / aw + 1e-8) * 5.0
    t_h = jnp.log(tgt4[3:4, :] / ah + 1e-8) * 5.0

    def sl1(d):
        return jnp.where(d < 1.0, 0.5 * d * d, d - 0.5)

    sl1_row = (sl1(jnp.abs(p[0:1, :] - t_cx)) + sl1(jnp.abs(p[1:2, :] - t_cy))
               + sl1(jnp.abs(p[2:3, :] - t_w)) + sl1(jnp.abs(p[3:4, :] - t_h)))
    sl1_tile = jnp.sum(sl1_row * fg_f)

    x = cls_ref[...]
    u = jnp.exp(-jnp.abs(x))
    opu = 1.0 + u
    lg = jnp.log(opu)
    sp = jnp.maximum(x, 0.0) + lg
    r = 1.0 / opu
    s = jnp.where(x >= 0.0, r, u * r)
    l0 = sp * (s * s)
    l1 = (sp - x) * ((1.0 - s) * (1.0 - s))

    term1 = jnp.sum(jax.lax.dot_general(cm_row, l0, _DN,
                                        preferred_element_type=jnp.float32))
    p0 = jax.lax.dot_general(oh32, l0, _DN,
                             preferred_element_type=jnp.float32)
    p1 = jax.lax.dot_general(oh32, l1, _DN,
                             preferred_element_type=jnp.float32)
    labs = labels_ref[...]
    cls_ids = jax.lax.broadcasted_iota(jnp.int32, (_M, _C), 1)
    labmat = (labs == cls_ids.astype(jnp.float32) + 1.0).astype(jnp.float32)
    corr1 = jnp.sum(p1 * labmat)
    corr0 = jnp.sum(p0 * labmat)
    focal_tile = 0.25 * (term1 - corr0) + 0.75 * corr1

    acc[0] += jnp.sum(fg_f)
    acc[1] += sl1_tile
    acc[2] += focal_tile

    @pl.when(jnp.logical_and(b == 0, j == 0))
    def _():
        out_ref[0] = 0.0
        out_ref[1] = 0.0

    @pl.when(j == _NT - 1)
    def _():
        n_fg = acc[0]
        bb_loss = jnp.where(n_fg > 0.0,
                            acc[1] / jnp.maximum(n_fg * 4.0, 1.0), 0.0)
        cls_loss = acc[2] / jnp.maximum(n_fg, 1.0)
        out_ref[0] += cls_loss / _B
        out_ref[1] += bb_loss / _B


def kernel(cls_preds, bbox_preds, anchors, boxes, labels):
    bbox_t = jnp.transpose(bbox_preds.reshape(_B, _NT, _T, 4), (0, 1, 3, 2))
    anchors_t = jnp.transpose(anchors.reshape(_B, _NT, _T, 4), (0, 1, 3, 2))
    boxes_t = jnp.transpose(boxes, (0, 2, 1))
    labels_f = labels.astype(jnp.float32).reshape(_B, _M, 1)

    out = pl.pallas_call(
        _loss_body,
        grid=(_B, _NT),
        in_specs=[
            pl.BlockSpec((None, _T, _C), lambda b, j: (b, j, 0)),
            pl.BlockSpec((None, None, 4, _T), lambda b, j: (b, j, 0, 0)),
            pl.BlockSpec((None, None, 4, _T), lambda b, j: (b, j, 0, 0)),
            pl.BlockSpec((None, _M, 4), lambda b, j: (b, 0, 0)),
            pl.BlockSpec((None, 4, _M), lambda b, j: (b, 0, 0)),
            pl.BlockSpec((None, _M, 1), lambda b, j: (b, 0, 0)),
        ],
        out_specs=pl.BlockSpec(memory_space=pltpu.SMEM),
        out_shape=jax.ShapeDtypeStruct((2,), jnp.float32),
        scratch_shapes=[pltpu.SMEM((3,), jnp.float32)],
    )(cls_preds, bbox_t, anchors_t, boxes, boxes_t, labels_f)
    return out
